# baseline (speedup 1.0000x reference)
import functools

import jax
import jax.numpy as jnp
from jax.experimental import pallas as pl
from jax.experimental.pallas import tpu as pltpu

_MARGIN = 0.5
_S = 64.0
_B = 1024
_C = 10000
_K = 3
_F = _C * _K
_LB = 768
_NSTEPS = -(-_F // _LB)


def _body(lab3_ref, x_ref, out_ref, m_ref, a_ref, lab_ref):
    j = pl.program_id(0)
    x = x_ref[...]
    lane = jax.lax.broadcasted_iota(jnp.int32, (_B, _LB), 1)
    flat = lane + j * _LB
    x1 = pltpu.roll(x, _LB - 1, 1)
    x2 = pltpu.roll(x, _LB - 2, 1)
    mx = jnp.maximum(jnp.maximum(x, x1), x2)
    mn = jnp.minimum(jnp.minimum(x, x1), x2)

    is_class = (lane % 3 == 0) & (flat < _F)
    is_label = flat == lab3_ref[...]

    cm = jnp.float32(jnp.cos(_MARGIN))
    sm = jnp.float32(jnp.sin(_MARGIN))
    margin = mn * cm - jnp.sqrt(jnp.maximum(1.0 - mn * mn, 0.0)) * sm
    logit = _S * jnp.where(is_label, margin, mx)

    neg = jnp.float32(-1e30)
    blk_max = jnp.max(jnp.where(is_class, logit, neg), axis=1, keepdims=True)

    @pl.when(j == 0)
    def _():
        m_ref[...] = jnp.full((_B, 1), neg, jnp.float32)
        a_ref[...] = jnp.zeros((_B, 1), jnp.float32)
        lab_ref[...] = jnp.zeros((_B, 1), jnp.float32)

    m_prev = m_ref[...]
    m_new = jnp.maximum(m_prev, blk_max)
    e = jnp.where(is_class, jnp.exp(logit - m_new), 0.0)
    a_ref[...] = a_ref[...] * jnp.exp(m_prev - m_new) + jnp.sum(
        e, axis=1, keepdims=True
    )
    m_ref[...] = m_new
    lab_ref[...] += jnp.sum(jnp.where(is_label, logit, 0.0), axis=1, keepdims=True)

    @pl.when(j == _NSTEPS - 1)
    def _():
        nll = jnp.log(a_ref[...]) + m_ref[...] - lab_ref[...]
        out_ref[0, 0] = jnp.sum(nll) * jnp.float32(1.0 / _B)


@jax.jit
def _run(x, lab3):
    return pl.pallas_call(
        _body,
        grid=(_NSTEPS,),
        in_specs=[
            pl.BlockSpec((_B, 1), lambda j: (0, 0)),
            pl.BlockSpec((_B, _LB), lambda j: (0, j)),
        ],
        out_specs=pl.BlockSpec((1, 1), lambda j: (0, 0), memory_space=pltpu.SMEM),
        out_shape=jax.ShapeDtypeStruct((1, 1), jnp.float32),
        scratch_shapes=[
            pltpu.VMEM((_B, 1), jnp.float32),
            pltpu.VMEM((_B, 1), jnp.float32),
            pltpu.VMEM((_B, 1), jnp.float32),
        ],
    )(lab3, x)


def kernel(costh, label):
    x = costh.reshape(_B, _F)
    lab3 = (label.astype(jnp.int32) * _K).reshape(_B, 1)
    return _run(x, lab3)[0, 0]


# speedup vs baseline: 12.0256x; 12.0256x over previous
import jax
import jax.numpy as jnp
from jax.experimental import pallas as pl
from jax.experimental.pallas import tpu as pltpu

_MARGIN = 0.5
_S = 64.0
_B = 1024
_C = 10000
_K = 3
_CB = 400
_NSTEPS = _C // _CB


def _body(lab_ref, x_ref, out_ref, m_ref, a_ref, lab_acc_ref):
    j = pl.program_id(0)
    x0 = x_ref[0]
    x1 = x_ref[1]
    x2 = x_ref[2]
    mx = jnp.maximum(jnp.maximum(x0, x1), x2)
    mn = jnp.minimum(jnp.minimum(x0, x1), x2)

    cls = jax.lax.broadcasted_iota(jnp.int32, (_CB, _B), 0) + j * _CB
    is_label = cls == lab_ref[...]

    cm = jnp.float32(jnp.cos(_MARGIN))
    sm = jnp.float32(jnp.sin(_MARGIN))
    margin = mn * cm - jnp.sqrt(jnp.maximum(1.0 - mn * mn, 0.0)) * sm
    logit = _S * jnp.where(is_label, margin, mx)

    blk_max = jnp.max(logit, axis=0, keepdims=True)

    @pl.when(j == 0)
    def _():
        m_ref[...] = jnp.full((1, _B), -1e30, jnp.float32)
        a_ref[...] = jnp.zeros((1, _B), jnp.float32)
        lab_acc_ref[...] = jnp.zeros((1, _B), jnp.float32)

    m_prev = m_ref[...]
    m_new = jnp.maximum(m_prev, blk_max)
    e = jnp.exp(logit - m_new)
    a_ref[...] = a_ref[...] * jnp.exp(m_prev - m_new) + jnp.sum(
        e, axis=0, keepdims=True
    )
    m_ref[...] = m_new
    lab_acc_ref[...] += jnp.sum(
        jnp.where(is_label, logit, 0.0), axis=0, keepdims=True
    )

    @pl.when(j == _NSTEPS - 1)
    def _():
        nll = jnp.log(a_ref[...]) + m_ref[...] - lab_acc_ref[...]
        out_ref[0, 0] = jnp.sum(nll) * jnp.float32(1.0 / _B)


@jax.jit
def _run(xt, lab):
    return pl.pallas_call(
        _body,
        grid=(_NSTEPS,),
        in_specs=[
            pl.BlockSpec((1, _B), lambda j: (0, 0)),
            pl.BlockSpec((_K, _CB, _B), lambda j: (0, j, 0)),
        ],
        out_specs=pl.BlockSpec((1, 1), lambda j: (0, 0), memory_space=pltpu.SMEM),
        out_shape=jax.ShapeDtypeStruct((1, 1), jnp.float32),
        scratch_shapes=[
            pltpu.VMEM((1, _B), jnp.float32),
            pltpu.VMEM((1, _B), jnp.float32),
            pltpu.VMEM((1, _B), jnp.float32),
        ],
    )(lab, xt)


def kernel(costh, label):
    xt = jnp.transpose(costh, (2, 1, 0))
    lab = label.astype(jnp.int32).reshape(1, _B)
    return _run(xt, lab)[0, 0]


# speedup vs baseline: 15.9121x; 1.3232x over previous
import jax
import jax.numpy as jnp
from jax.experimental import pallas as pl
from jax.experimental.pallas import tpu as pltpu

_MARGIN = 0.5
_S = 64.0
_B = 1024
_C = 10000
_K = 3
_CB = 400
_NSTEPS = _C // _CB
_LOG2E = 1.4426950408889634
_KS = _S * _LOG2E
_NEG = -4.0


def _body(lab_ref, x_ref, out_ref, m_ref, a_ref, mn_ref):
    j = pl.program_id(0)
    x0 = x_ref[0]
    x1 = x_ref[1]
    x2 = x_ref[2]
    mx = jnp.maximum(jnp.maximum(x0, x1), x2)
    mn = jnp.minimum(jnp.minimum(x0, x1), x2)

    row = jax.lax.broadcasted_iota(jnp.int32, (_CB, _B), 0)
    is_label = row == lab_ref[...] - j * _CB

    mxs = jnp.where(is_label, jnp.float32(_NEG), mx)
    blk_max = jnp.max(mxs, axis=0, keepdims=True)

    @pl.when(j == 0)
    def _():
        m_ref[...] = jnp.full((1, _B), _NEG, jnp.float32)
        a_ref[...] = jnp.zeros((1, _B), jnp.float32)
        mn_ref[...] = jnp.zeros((1, _B), jnp.float32)

    m_prev = m_ref[...]
    m_new = jnp.maximum(m_prev, blk_max)
    c = _KS * m_new
    e = jnp.exp2(_KS * mxs - c)
    a_ref[...] = a_ref[...] * jnp.exp2(_KS * m_prev - c) + jnp.sum(
        e, axis=0, keepdims=True
    )
    m_ref[...] = m_new
    mn_ref[...] += jnp.sum(jnp.where(is_label, mn, 0.0), axis=0, keepdims=True)

    @pl.when(j == _NSTEPS - 1)
    def _():
        cm = jnp.float32(jnp.cos(_MARGIN))
        sm = jnp.float32(jnp.sin(_MARGIN))
        v = mn_ref[...]
        m_lab = _S * (v * cm - jnp.sqrt(jnp.maximum(1.0 - v * v, 0.0)) * sm)
        m_all = jnp.maximum(_S * m_ref[...], m_lab)
        a_all = a_ref[...] * jnp.exp(_S * m_ref[...] - m_all) + jnp.exp(
            m_lab - m_all
        )
        nll = jnp.log(a_all) + m_all - m_lab
        out_ref[0, 0] = jnp.sum(nll) * jnp.float32(1.0 / _B)


@jax.jit
def _run(xt, lab):
    return pl.pallas_call(
        _body,
        grid=(_NSTEPS,),
        in_specs=[
            pl.BlockSpec((1, _B), lambda j: (0, 0)),
            pl.BlockSpec((_K, _CB, _B), lambda j: (0, j, 0)),
        ],
        out_specs=pl.BlockSpec((1, 1), lambda j: (0, 0), memory_space=pltpu.SMEM),
        out_shape=jax.ShapeDtypeStruct((1, 1), jnp.float32),
        scratch_shapes=[
            pltpu.VMEM((1, _B), jnp.float32),
            pltpu.VMEM((1, _B), jnp.float32),
            pltpu.VMEM((1, _B), jnp.float32),
        ],
    )(lab, xt)


def kernel(costh, label):
    xt = jnp.transpose(costh, (2, 1, 0))
    lab = label.astype(jnp.int32).reshape(1, _B)
    return _run(xt, lab)[0, 0]
